# Initial kernel scaffold; baseline (speedup 1.0000x reference)
import jax
import jax.numpy as jnp
from jax.experimental import pallas as pl


def kernel(x, W1, b1, g1, be1, W2, b2, g2, be2, W3, b3, g3, be3, Wg1, bg1, gg1, beg1, Wg2, bg2, gg2, beg2, W4, b4, g4, be4):
    raise NotImplementedError("write your pallas kernel here")


# speedup vs baseline: 3.1346x; 3.1346x over previous
import jax
import jax.numpy as jnp
from jax.experimental import pallas as pl

F32 = jnp.float32
EPS = 1e-5
K = 16
BM = 256


def _dot(a, b, contract_a=1, contract_b=1):
    return jax.lax.dot_general(
        a, b, (((contract_a,), (contract_b,)), ((), ())),
        precision=jax.lax.Precision.HIGHEST,
        preferred_element_type=F32)


def _dotb(a, b, contract_a=1, contract_b=1):
    return jax.lax.dot_general(
        a.astype(jnp.bfloat16), b.astype(jnp.bfloat16),
        (((contract_a,), (contract_b,)), ((), ())),
        preferred_element_type=F32)


def _rownorm_row(h):
    ones = jnp.ones((1, h.shape[1]), F32)
    return _dot(ones, h * h)


def _topk_gathermax(d, h_all, n):
    iota = jax.lax.broadcasted_iota(jnp.int32, d.shape, 1)
    acc = jnp.full((d.shape[0], h_all.shape[1]), -jnp.inf, F32)
    for _ in range(K):
        v = jnp.min(d, axis=1, keepdims=True)
        first = jnp.min(jnp.where(d <= v, iota, n), axis=1, keepdims=True)
        onehot = iota == first
        d = jnp.where(onehot, jnp.inf, d)
        g = _dot(onehot.astype(F32), h_all, 1, 0)
        acc = jnp.maximum(acc, g)
    return acc


def _is_first():
    return (pl.program_id(0) == 0) & (pl.program_id(1) == 0)


def _acc_moments(mom_ref, y):
    @pl.when(_is_first())
    def _():
        mom_ref[...] = jnp.zeros_like(mom_ref)
    ones = jnp.ones((1, y.shape[0]), F32)
    mom_ref[0:1, :] += _dot(ones, y, 1, 0)
    mom_ref[1:2, :] += _dot(ones, y * y, 1, 0)


def _k1_body(xb_ref, xa_ref, w_ref, b_ref, y_ref, mom_ref):
    n = xa_ref.shape[1]
    xb = xb_ref[0]
    xa = xa_ref[0]
    nb = jnp.sum(xb * xb, axis=1, keepdims=True)
    na = _rownorm_row(xa)
    d = nb + na - 2.0 * _dotb(xb, xa)
    iota = jax.lax.broadcasted_iota(jnp.int32, d.shape, 1)
    gs = []
    for _ in range(K):
        v = jnp.min(d, axis=1, keepdims=True)
        first = jnp.min(jnp.where(d <= v, iota, n), axis=1, keepdims=True)
        onehot = iota == first
        d = jnp.where(onehot, jnp.inf, d)
        gs.append(_dot(onehot.astype(F32), xa, 1, 0))
    vs = list(gs)
    while len(vs) > 1:
        vs = [vs[i] + vs[i + 1] for i in range(0, len(vs), 2)]
    mean = vs[0] * (1.0 / K)
    cov = jnp.zeros((xb.shape[0], 9), F32)
    for g in gs:
        kc = (g - mean).astype(jnp.bfloat16).astype(F32)
        left = jnp.concatenate(
            [kc[:, a:a + 1] for a in (0, 0, 0, 1, 1, 1, 2, 2, 2)], axis=1)
        right = jnp.concatenate([kc, kc, kc], axis=1)
        cov = cov + left * right
    h0 = jnp.concatenate([xb, cov], axis=1)
    y = _dotb(h0, w_ref[...]) + b_ref[...]
    y_ref[0] = y
    _acc_moments(mom_ref, y)


def _mlp_body(y_ref, sc_ref, sh_ref, w_ref, b_ref, o_ref, mom_ref):
    h = jax.nn.relu(y_ref[0] * sc_ref[...] + sh_ref[...])
    y = _dotb(h, w_ref[...]) + b_ref[...]
    o_ref[0] = y
    _acc_moments(mom_ref, y)


def _graph_body(y_ref, yb_ref, sc_ref, sh_ref, w_ref, b_ref, o_ref, mom_ref):
    n = y_ref.shape[1]
    h_all = jax.nn.relu(y_ref[0] * sc_ref[...] + sh_ref[...])
    hb = jax.nn.relu(yb_ref[0] * sc_ref[...] + sh_ref[...])
    nb = jnp.sum(hb * hb, axis=1, keepdims=True)
    na = _rownorm_row(h_all)
    d = nb + na - 2.0 * _dotb(hb, h_all)
    agg = _topk_gathermax(d, h_all, n)
    y = _dotb(agg, w_ref[...]) + b_ref[...]
    o_ref[0] = y
    _acc_moments(mom_ref, y)


def _final_body(y_ref, sc_ref, sh_ref, w_ref, b_ref, mx_ref, mn_ref, mom_ref):
    h = jax.nn.relu(y_ref[0] * sc_ref[...] + sh_ref[...])
    y = _dotb(h, w_ref[...]) + b_ref[...]

    @pl.when(pl.program_id(1) == 0)
    def _():
        mx_ref[...] = jnp.full_like(mx_ref, -jnp.inf)
        mn_ref[...] = jnp.full_like(mn_ref, jnp.inf)
    mx_ref[0] = jnp.maximum(mx_ref[0], jnp.max(y, axis=0, keepdims=True))
    mn_ref[0] = jnp.minimum(mn_ref[0], jnp.min(y, axis=0, keepdims=True))
    _acc_moments(mom_ref, y)


def _stats(mom, g, be, cnt):
    mean = mom[0] / cnt
    var = jnp.maximum(mom[1] / cnt - mean * mean, 0.0)
    scale = g / jnp.sqrt(var + EPS)
    shift = be - mean * scale
    return scale[None, :], shift[None, :]


@jax.jit
def kernel(x, W1, b1, g1, be1, W2, b2, g2, be2, W3, b3, g3, be3,
           Wg1, bg1, gg1, beg1, Wg2, bg2, gg2, beg2, W4, b4, g4, be4):
    B, N, _ = x.shape
    nb = N // BM
    grid = (B, nb)
    cnt = B * N

    def row2(v):
        return v[None, :]

    def run(body, ins, in_specs, outs, out_specs):
        return pl.pallas_call(
            body, grid=grid,
            in_specs=in_specs,
            out_specs=out_specs,
            out_shape=outs,
        )(*ins)

    def spec_blk(c):
        return pl.BlockSpec((1, BM, c), lambda b, i: (b, i, 0))

    def spec_full(n, c):
        return pl.BlockSpec((1, n, c), lambda b, i: (b, 0, 0))

    def spec_w(r, c):
        return pl.BlockSpec((r, c), lambda b, i: (0, 0))

    def spec_mom(c):
        return pl.BlockSpec((2, c), lambda b, i: (0, 0))

    y1, mom1 = run(
        _k1_body,
        [x, x, W1, row2(b1)],
        [spec_blk(3), spec_full(N, 3), spec_w(64, 12), spec_w(1, 64)],
        [jax.ShapeDtypeStruct((B, N, 64), F32),
         jax.ShapeDtypeStruct((2, 64), F32)],
        [spec_blk(64), spec_mom(64)])
    sc1, sh1 = _stats(mom1, g1, be1, cnt)

    y2, mom2 = run(
        _mlp_body,
        [y1, sc1, sh1, W2, row2(b2)],
        [spec_blk(64), spec_w(1, 64), spec_w(1, 64), spec_w(64, 64),
         spec_w(1, 64)],
        [jax.ShapeDtypeStruct((B, N, 64), F32),
         jax.ShapeDtypeStruct((2, 64), F32)],
        [spec_blk(64), spec_mom(64)])
    sc2, sh2 = _stats(mom2, g2, be2, cnt)

    y3, mom3 = run(
        _mlp_body,
        [y2, sc2, sh2, W3, row2(b3)],
        [spec_blk(64), spec_w(1, 64), spec_w(1, 64), spec_w(64, 64),
         spec_w(1, 64)],
        [jax.ShapeDtypeStruct((B, N, 64), F32),
         jax.ShapeDtypeStruct((2, 64), F32)],
        [spec_blk(64), spec_mom(64)])
    sc3, sh3 = _stats(mom3, g3, be3, cnt)

    yg1, momg1 = run(
        _graph_body,
        [y3, y3, sc3, sh3, Wg1, row2(bg1)],
        [spec_full(N, 64), spec_blk(64), spec_w(1, 64), spec_w(1, 64), spec_w(128, 64),
         spec_w(1, 128)],
        [jax.ShapeDtypeStruct((B, N, 128), F32),
         jax.ShapeDtypeStruct((2, 128), F32)],
        [spec_blk(128), spec_mom(128)])
    scg1, shg1 = _stats(momg1, gg1, beg1, cnt)

    yg2, momg2 = run(
        _graph_body,
        [yg1, yg1, scg1, shg1, Wg2, row2(bg2)],
        [spec_full(N, 128), spec_blk(128), spec_w(1, 128), spec_w(1, 128),
         spec_w(1024, 128), spec_w(1, 1024)],
        [jax.ShapeDtypeStruct((B, N, 1024), F32),
         jax.ShapeDtypeStruct((2, 1024), F32)],
        [spec_blk(1024), spec_mom(1024)])
    scg2, shg2 = _stats(momg2, gg2, beg2, cnt)

    mx, mn, mom4 = run(
        _final_body,
        [yg2, scg2, shg2, W4, row2(b4)],
        [spec_blk(1024), spec_w(1, 1024), spec_w(1, 1024),
         spec_w(512, 1024), spec_w(1, 512)],
        [jax.ShapeDtypeStruct((B, 1, 512), F32),
         jax.ShapeDtypeStruct((B, 1, 512), F32),
         jax.ShapeDtypeStruct((2, 512), F32)],
        [pl.BlockSpec((1, 1, 512), lambda b, i: (b, 0, 0)),
         pl.BlockSpec((1, 1, 512), lambda b, i: (b, 0, 0)),
         spec_mom(512)])
    mx = mx[:, 0, :]
    mn = mn[:, 0, :]
    sc4, sh4 = _stats(mom4, g4, be4, cnt)

    out = jnp.where(sc4 >= 0, mx * sc4, mn * sc4) + sh4
    return out


# speedup vs baseline: 6.1644x; 1.9666x over previous
import jax
import jax.numpy as jnp
from jax.experimental import pallas as pl

F32 = jnp.float32
EPS = 1e-5
K = 16
BM = 256


def _dot(a, b, contract_a=1, contract_b=1):
    return jax.lax.dot_general(
        a, b, (((contract_a,), (contract_b,)), ((), ())),
        precision=jax.lax.Precision.HIGHEST,
        preferred_element_type=F32)


def _dotb(a, b, contract_a=1, contract_b=1):
    return jax.lax.dot_general(
        a.astype(jnp.bfloat16), b.astype(jnp.bfloat16),
        (((contract_a,), (contract_b,)), ((), ())),
        preferred_element_type=F32)


def _split3(h):
    h1 = h.astype(jnp.bfloat16)
    r1 = h - h1.astype(F32)
    h2 = r1.astype(jnp.bfloat16)
    h3 = (r1 - h2.astype(F32)).astype(jnp.bfloat16)
    return (h1, h2, h3)


def _dot_raw(a, b, contract_a=1, contract_b=1):
    return jax.lax.dot_general(
        a, b, (((contract_a,), (contract_b,)), ((), ())),
        preferred_element_type=F32)


def _gather_dot(onehot_bf16, splits):
    g = _dot_raw(onehot_bf16, splits[0], 1, 0)
    g = g + _dot_raw(onehot_bf16, splits[1], 1, 0)
    g = g + _dot_raw(onehot_bf16, splits[2], 1, 0)
    return g


def _rownorm_row(h):
    ones = jnp.ones((1, h.shape[1]), F32)
    return _dot(ones, h * h)


def _topk_gathermax(d, h_all, n):
    iota = jax.lax.broadcasted_iota(jnp.int32, d.shape, 1)
    acc = jnp.full((d.shape[0], h_all.shape[1]), -jnp.inf, F32)
    splits = _split3(h_all)
    for _ in range(K):
        v = jnp.min(d, axis=1, keepdims=True)
        first = jnp.min(jnp.where(d <= v, iota, n), axis=1, keepdims=True)
        onehot = iota == first
        d = jnp.where(onehot, jnp.inf, d)
        g = _gather_dot(onehot.astype(jnp.bfloat16), splits)
        acc = jnp.maximum(acc, g)
    return acc


def _is_first():
    return (pl.program_id(0) == 0) & (pl.program_id(1) == 0)


def _acc_moments(mom_ref, y):
    @pl.when(_is_first())
    def _():
        mom_ref[...] = jnp.zeros_like(mom_ref)
    ones = jnp.ones((1, y.shape[0]), F32)
    mom_ref[0:1, :] += _dot(ones, y, 1, 0)
    mom_ref[1:2, :] += _dot(ones, y * y, 1, 0)


def _k1_body(xb_ref, xa_ref, w_ref, b_ref, y_ref, mom_ref):
    n = xa_ref.shape[1]
    xb = xb_ref[0]
    xa = xa_ref[0]
    nb = jnp.sum(xb * xb, axis=1, keepdims=True)
    na = _rownorm_row(xa)
    d = nb + na - 2.0 * _dotb(xb, xa)
    iota = jax.lax.broadcasted_iota(jnp.int32, d.shape, 1)
    splits = _split3(xa)
    gs = []
    for _ in range(K):
        v = jnp.min(d, axis=1, keepdims=True)
        first = jnp.min(jnp.where(d <= v, iota, n), axis=1, keepdims=True)
        onehot = iota == first
        d = jnp.where(onehot, jnp.inf, d)
        gs.append(_gather_dot(onehot.astype(jnp.bfloat16), splits))
    vs = list(gs)
    while len(vs) > 1:
        vs = [vs[i] + vs[i + 1] for i in range(0, len(vs), 2)]
    mean = vs[0] * (1.0 / K)
    cov = jnp.zeros((xb.shape[0], 9), F32)
    for g in gs:
        kc = (g - mean).astype(jnp.bfloat16).astype(F32)
        left = jnp.concatenate(
            [kc[:, a:a + 1] for a in (0, 0, 0, 1, 1, 1, 2, 2, 2)], axis=1)
        right = jnp.concatenate([kc, kc, kc], axis=1)
        cov = cov + left * right
    h0 = jnp.concatenate([xb, cov], axis=1)
    y = _dotb(h0, w_ref[...]) + b_ref[...]
    y_ref[0] = y
    _acc_moments(mom_ref, y)


def _mlp_body(y_ref, sc_ref, sh_ref, w_ref, b_ref, o_ref, mom_ref):
    h = jax.nn.relu(y_ref[0] * sc_ref[...] + sh_ref[...])
    y = _dotb(h, w_ref[...]) + b_ref[...]
    o_ref[0] = y
    _acc_moments(mom_ref, y)


def _graph_body(y_ref, yb_ref, sc_ref, sh_ref, w_ref, b_ref, o_ref, mom_ref):
    n = y_ref.shape[1]
    h_all = jax.nn.relu(y_ref[0] * sc_ref[...] + sh_ref[...])
    hb = jax.nn.relu(yb_ref[0] * sc_ref[...] + sh_ref[...])
    nb = jnp.sum(hb * hb, axis=1, keepdims=True)
    na = _rownorm_row(h_all)
    d = nb + na - 2.0 * _dotb(hb, h_all)
    agg = _topk_gathermax(d, h_all, n)
    y = _dotb(agg, w_ref[...]) + b_ref[...]
    o_ref[0] = y
    _acc_moments(mom_ref, y)


def _final_body(y_ref, sc_ref, sh_ref, w_ref, b_ref, mx_ref, mn_ref, mom_ref):
    h = jax.nn.relu(y_ref[0] * sc_ref[...] + sh_ref[...])
    y = _dotb(h, w_ref[...]) + b_ref[...]

    @pl.when(pl.program_id(1) == 0)
    def _():
        mx_ref[...] = jnp.full_like(mx_ref, -jnp.inf)
        mn_ref[...] = jnp.full_like(mn_ref, jnp.inf)
    mx_ref[0] = jnp.maximum(mx_ref[0], jnp.max(y, axis=0, keepdims=True))
    mn_ref[0] = jnp.minimum(mn_ref[0], jnp.min(y, axis=0, keepdims=True))
    _acc_moments(mom_ref, y)


def _stats(mom, g, be, cnt):
    mean = mom[0] / cnt
    var = jnp.maximum(mom[1] / cnt - mean * mean, 0.0)
    scale = g / jnp.sqrt(var + EPS)
    shift = be - mean * scale
    return scale[None, :], shift[None, :]


@jax.jit
def kernel(x, W1, b1, g1, be1, W2, b2, g2, be2, W3, b3, g3, be3,
           Wg1, bg1, gg1, beg1, Wg2, bg2, gg2, beg2, W4, b4, g4, be4):
    B, N, _ = x.shape
    nb = N // BM
    grid = (B, nb)
    cnt = B * N

    def row2(v):
        return v[None, :]

    def run(body, ins, in_specs, outs, out_specs):
        return pl.pallas_call(
            body, grid=grid,
            in_specs=in_specs,
            out_specs=out_specs,
            out_shape=outs,
        )(*ins)

    def spec_blk(c):
        return pl.BlockSpec((1, BM, c), lambda b, i: (b, i, 0))

    def spec_full(n, c):
        return pl.BlockSpec((1, n, c), lambda b, i: (b, 0, 0))

    def spec_w(r, c):
        return pl.BlockSpec((r, c), lambda b, i: (0, 0))

    def spec_mom(c):
        return pl.BlockSpec((2, c), lambda b, i: (0, 0))

    y1, mom1 = run(
        _k1_body,
        [x, x, W1, row2(b1)],
        [spec_blk(3), spec_full(N, 3), spec_w(64, 12), spec_w(1, 64)],
        [jax.ShapeDtypeStruct((B, N, 64), F32),
         jax.ShapeDtypeStruct((2, 64), F32)],
        [spec_blk(64), spec_mom(64)])
    sc1, sh1 = _stats(mom1, g1, be1, cnt)

    y2, mom2 = run(
        _mlp_body,
        [y1, sc1, sh1, W2, row2(b2)],
        [spec_blk(64), spec_w(1, 64), spec_w(1, 64), spec_w(64, 64),
         spec_w(1, 64)],
        [jax.ShapeDtypeStruct((B, N, 64), F32),
         jax.ShapeDtypeStruct((2, 64), F32)],
        [spec_blk(64), spec_mom(64)])
    sc2, sh2 = _stats(mom2, g2, be2, cnt)

    y3, mom3 = run(
        _mlp_body,
        [y2, sc2, sh2, W3, row2(b3)],
        [spec_blk(64), spec_w(1, 64), spec_w(1, 64), spec_w(64, 64),
         spec_w(1, 64)],
        [jax.ShapeDtypeStruct((B, N, 64), F32),
         jax.ShapeDtypeStruct((2, 64), F32)],
        [spec_blk(64), spec_mom(64)])
    sc3, sh3 = _stats(mom3, g3, be3, cnt)

    yg1, momg1 = run(
        _graph_body,
        [y3, y3, sc3, sh3, Wg1, row2(bg1)],
        [spec_full(N, 64), spec_blk(64), spec_w(1, 64), spec_w(1, 64), spec_w(128, 64),
         spec_w(1, 128)],
        [jax.ShapeDtypeStruct((B, N, 128), F32),
         jax.ShapeDtypeStruct((2, 128), F32)],
        [spec_blk(128), spec_mom(128)])
    scg1, shg1 = _stats(momg1, gg1, beg1, cnt)

    yg2, momg2 = run(
        _graph_body,
        [yg1, yg1, scg1, shg1, Wg2, row2(bg2)],
        [spec_full(N, 128), spec_blk(128), spec_w(1, 128), spec_w(1, 128),
         spec_w(1024, 128), spec_w(1, 1024)],
        [jax.ShapeDtypeStruct((B, N, 1024), F32),
         jax.ShapeDtypeStruct((2, 1024), F32)],
        [spec_blk(1024), spec_mom(1024)])
    scg2, shg2 = _stats(momg2, gg2, beg2, cnt)

    mx, mn, mom4 = run(
        _final_body,
        [yg2, scg2, shg2, W4, row2(b4)],
        [spec_blk(1024), spec_w(1, 1024), spec_w(1, 1024),
         spec_w(512, 1024), spec_w(1, 512)],
        [jax.ShapeDtypeStruct((B, 1, 512), F32),
         jax.ShapeDtypeStruct((B, 1, 512), F32),
         jax.ShapeDtypeStruct((2, 512), F32)],
        [pl.BlockSpec((1, 1, 512), lambda b, i: (b, 0, 0)),
         pl.BlockSpec((1, 1, 512), lambda b, i: (b, 0, 0)),
         spec_mom(512)])
    mx = mx[:, 0, :]
    mn = mn[:, 0, :]
    sc4, sh4 = _stats(mom4, g4, be4, cnt)

    out = jnp.where(sc4 >= 0, mx * sc4, mn * sc4) + sh4
    return out


# speedup vs baseline: 6.3434x; 1.0290x over previous
import jax
import jax.numpy as jnp
from jax.experimental import pallas as pl

F32 = jnp.float32
EPS = 1e-5
K = 16
BM = 256


def _dot(a, b, contract_a=1, contract_b=1):
    return jax.lax.dot_general(
        a, b, (((contract_a,), (contract_b,)), ((), ())),
        precision=jax.lax.Precision.HIGHEST,
        preferred_element_type=F32)


def _dotb(a, b, contract_a=1, contract_b=1):
    return jax.lax.dot_general(
        a.astype(jnp.bfloat16), b.astype(jnp.bfloat16),
        (((contract_a,), (contract_b,)), ((), ())),
        preferred_element_type=F32)


def _split3(h):
    h1 = h.astype(jnp.bfloat16)
    r1 = h - h1.astype(F32)
    h2 = r1.astype(jnp.bfloat16)
    h3 = (r1 - h2.astype(F32)).astype(jnp.bfloat16)
    return (h1, h2, h3)


def _dot_raw(a, b, contract_a=1, contract_b=1):
    return jax.lax.dot_general(
        a, b, (((contract_a,), (contract_b,)), ((), ())),
        preferred_element_type=F32)


def _gather_dot(onehot_bf16, splits):
    g = _dot_raw(onehot_bf16, splits[0], 1, 0)
    g = g + _dot_raw(onehot_bf16, splits[1], 1, 0)
    g = g + _dot_raw(onehot_bf16, splits[2], 1, 0)
    return g


def _rownorm_row(h):
    ones = jnp.ones((1, h.shape[1]), jnp.bfloat16)
    sq = _split3(h * h)
    return (_dot_raw(ones, sq[0]) + _dot_raw(ones, sq[1])
            + _dot_raw(ones, sq[2]))


def _topk_gathermax(d, h_all, n):
    iota = jax.lax.broadcasted_iota(jnp.int32, d.shape, 1)
    acc = jnp.full((d.shape[0], h_all.shape[1]), -jnp.inf, F32)
    splits = _split3(h_all)
    for _ in range(K):
        v = jnp.min(d, axis=1, keepdims=True)
        first = jnp.min(jnp.where(d <= v, iota, n), axis=1, keepdims=True)
        onehot = iota == first
        d = jnp.where(onehot, jnp.inf, d)
        g = _gather_dot(onehot.astype(jnp.bfloat16), splits)
        acc = jnp.maximum(acc, g)
    return acc


def _is_first():
    return (pl.program_id(0) == 0) & (pl.program_id(1) == 0)


def _acc_moments(mom_ref, y):
    @pl.when(_is_first())
    def _():
        mom_ref[...] = jnp.zeros_like(mom_ref)
    mom_ref[0:1, :] += jnp.sum(y, axis=0, keepdims=True)
    mom_ref[1:2, :] += jnp.sum(y * y, axis=0, keepdims=True)


def _k1_body(xb_ref, xa_ref, w_ref, b_ref, y_ref, mom_ref):
    n = xa_ref.shape[1]
    xb = xb_ref[0]
    xa = xa_ref[0]
    nb = jnp.sum(xb * xb, axis=1, keepdims=True)
    na = _rownorm_row(xa)
    d = nb + na - 2.0 * _dotb(xb, xa)
    iota = jax.lax.broadcasted_iota(jnp.int32, d.shape, 1)
    splits = _split3(xa)
    gs = []
    for _ in range(K):
        v = jnp.min(d, axis=1, keepdims=True)
        first = jnp.min(jnp.where(d <= v, iota, n), axis=1, keepdims=True)
        onehot = iota == first
        d = jnp.where(onehot, jnp.inf, d)
        gs.append(_gather_dot(onehot.astype(jnp.bfloat16), splits))
    vs = list(gs)
    while len(vs) > 1:
        vs = [vs[i] + vs[i + 1] for i in range(0, len(vs), 2)]
    mean = vs[0] * (1.0 / K)
    cov = jnp.zeros((xb.shape[0], 9), F32)
    for g in gs:
        kc = (g - mean).astype(jnp.bfloat16).astype(F32)
        left = jnp.concatenate(
            [kc[:, a:a + 1] for a in (0, 0, 0, 1, 1, 1, 2, 2, 2)], axis=1)
        right = jnp.concatenate([kc, kc, kc], axis=1)
        cov = cov + left * right
    h0 = jnp.concatenate([xb, cov], axis=1)
    y = _dotb(h0, w_ref[...]) + b_ref[...]
    y_ref[0] = y
    _acc_moments(mom_ref, y)


def _mlp_body(y_ref, sc_ref, sh_ref, w_ref, b_ref, o_ref, mom_ref):
    h = jax.nn.relu(y_ref[0] * sc_ref[...] + sh_ref[...])
    y = _dotb(h, w_ref[...]) + b_ref[...]
    o_ref[0] = y
    _acc_moments(mom_ref, y)


def _graph_body(y_ref, yb_ref, sc_ref, sh_ref, w_ref, b_ref, o_ref, mom_ref):
    n = y_ref.shape[1]
    h_all = jax.nn.relu(y_ref[0] * sc_ref[...] + sh_ref[...])
    hb = jax.nn.relu(yb_ref[0] * sc_ref[...] + sh_ref[...])
    nb = jnp.sum(hb * hb, axis=1, keepdims=True)
    na = _rownorm_row(h_all)
    d = nb + na - 2.0 * _dotb(hb, h_all)
    agg = _topk_gathermax(d, h_all, n)
    y = _dotb(agg, w_ref[...]) + b_ref[...]
    o_ref[0] = y
    _acc_moments(mom_ref, y)


def _final_body(y_ref, sc_ref, sh_ref, w_ref, b_ref, mx_ref, mn_ref, mom_ref):
    h = jax.nn.relu(y_ref[0] * sc_ref[...] + sh_ref[...])
    y = _dotb(h, w_ref[...]) + b_ref[...]

    @pl.when(pl.program_id(1) == 0)
    def _():
        mx_ref[...] = jnp.full_like(mx_ref, -jnp.inf)
        mn_ref[...] = jnp.full_like(mn_ref, jnp.inf)
    mx_ref[0] = jnp.maximum(mx_ref[0], jnp.max(y, axis=0, keepdims=True))
    mn_ref[0] = jnp.minimum(mn_ref[0], jnp.min(y, axis=0, keepdims=True))
    _acc_moments(mom_ref, y)


def _stats(mom, g, be, cnt):
    mean = mom[0] / cnt
    var = jnp.maximum(mom[1] / cnt - mean * mean, 0.0)
    scale = g / jnp.sqrt(var + EPS)
    shift = be - mean * scale
    return scale[None, :], shift[None, :]


@jax.jit
def kernel(x, W1, b1, g1, be1, W2, b2, g2, be2, W3, b3, g3, be3,
           Wg1, bg1, gg1, beg1, Wg2, bg2, gg2, beg2, W4, b4, g4, be4):
    B, N, _ = x.shape
    nb = N // BM
    grid = (B, nb)
    cnt = B * N

    def row2(v):
        return v[None, :]

    def run(body, ins, in_specs, outs, out_specs):
        return pl.pallas_call(
            body, grid=grid,
            in_specs=in_specs,
            out_specs=out_specs,
            out_shape=outs,
        )(*ins)

    def spec_blk(c):
        return pl.BlockSpec((1, BM, c), lambda b, i: (b, i, 0))

    def spec_full(n, c):
        return pl.BlockSpec((1, n, c), lambda b, i: (b, 0, 0))

    def spec_w(r, c):
        return pl.BlockSpec((r, c), lambda b, i: (0, 0))

    def spec_mom(c):
        return pl.BlockSpec((2, c), lambda b, i: (0, 0))

    y1, mom1 = run(
        _k1_body,
        [x, x, W1, row2(b1)],
        [spec_blk(3), spec_full(N, 3), spec_w(64, 12), spec_w(1, 64)],
        [jax.ShapeDtypeStruct((B, N, 64), F32),
         jax.ShapeDtypeStruct((2, 64), F32)],
        [spec_blk(64), spec_mom(64)])
    sc1, sh1 = _stats(mom1, g1, be1, cnt)

    y2, mom2 = run(
        _mlp_body,
        [y1, sc1, sh1, W2, row2(b2)],
        [spec_blk(64), spec_w(1, 64), spec_w(1, 64), spec_w(64, 64),
         spec_w(1, 64)],
        [jax.ShapeDtypeStruct((B, N, 64), F32),
         jax.ShapeDtypeStruct((2, 64), F32)],
        [spec_blk(64), spec_mom(64)])
    sc2, sh2 = _stats(mom2, g2, be2, cnt)

    y3, mom3 = run(
        _mlp_body,
        [y2, sc2, sh2, W3, row2(b3)],
        [spec_blk(64), spec_w(1, 64), spec_w(1, 64), spec_w(64, 64),
         spec_w(1, 64)],
        [jax.ShapeDtypeStruct((B, N, 64), F32),
         jax.ShapeDtypeStruct((2, 64), F32)],
        [spec_blk(64), spec_mom(64)])
    sc3, sh3 = _stats(mom3, g3, be3, cnt)

    yg1, momg1 = run(
        _graph_body,
        [y3, y3, sc3, sh3, Wg1, row2(bg1)],
        [spec_full(N, 64), spec_blk(64), spec_w(1, 64), spec_w(1, 64), spec_w(128, 64),
         spec_w(1, 128)],
        [jax.ShapeDtypeStruct((B, N, 128), F32),
         jax.ShapeDtypeStruct((2, 128), F32)],
        [spec_blk(128), spec_mom(128)])
    scg1, shg1 = _stats(momg1, gg1, beg1, cnt)

    yg2, momg2 = run(
        _graph_body,
        [yg1, yg1, scg1, shg1, Wg2, row2(bg2)],
        [spec_full(N, 128), spec_blk(128), spec_w(1, 128), spec_w(1, 128),
         spec_w(1024, 128), spec_w(1, 1024)],
        [jax.ShapeDtypeStruct((B, N, 1024), F32),
         jax.ShapeDtypeStruct((2, 1024), F32)],
        [spec_blk(1024), spec_mom(1024)])
    scg2, shg2 = _stats(momg2, gg2, beg2, cnt)

    mx, mn, mom4 = run(
        _final_body,
        [yg2, scg2, shg2, W4, row2(b4)],
        [spec_blk(1024), spec_w(1, 1024), spec_w(1, 1024),
         spec_w(512, 1024), spec_w(1, 512)],
        [jax.ShapeDtypeStruct((B, 1, 512), F32),
         jax.ShapeDtypeStruct((B, 1, 512), F32),
         jax.ShapeDtypeStruct((2, 512), F32)],
        [pl.BlockSpec((1, 1, 512), lambda b, i: (b, 0, 0)),
         pl.BlockSpec((1, 1, 512), lambda b, i: (b, 0, 0)),
         spec_mom(512)])
    mx = mx[:, 0, :]
    mn = mn[:, 0, :]
    sc4, sh4 = _stats(mom4, g4, be4, cnt)

    out = jnp.where(sc4 >= 0, mx * sc4, mn * sc4) + sh4
    return out
